# Initial kernel scaffold; baseline (speedup 1.0000x reference)
import jax
import jax.numpy as jnp
from jax.experimental import pallas as pl


def kernel(node_feat, edge_feat, eig_vec, edge_index, W_pre, b_pre, W_post, b_post, W_gate, b_gate):
    raise NotImplementedError("write your pallas kernel here")


# speedup vs baseline: 1.1241x; 1.1241x over previous
import functools

import jax
import jax.numpy as jnp
from jax.experimental import pallas as pl


def _mm_kernel(x_ref, w_ref, o_ref):
    o_ref[...] = jnp.dot(x_ref[...], w_ref[...],
                         preferred_element_type=jnp.float32)


def _matmul(x, w, block_rows):
    m, k = x.shape
    _, n = w.shape
    grid = m // block_rows
    return pl.pallas_call(
        _mm_kernel,
        grid=(grid,),
        in_specs=[
            pl.BlockSpec((block_rows, k), lambda i: (i, 0)),
            pl.BlockSpec((k, n), lambda i: (0, 0)),
        ],
        out_specs=pl.BlockSpec((block_rows, n), lambda i: (i, 0)),
        out_shape=jax.ShapeDtypeStruct((m, n), jnp.float32),
    )(x, w)


def _pool_kernel(h_ref, wg_ref, o_ref):
    h = h_ref[...]
    g = jnp.sum(h * wg_ref[...], axis=1, keepdims=True)
    g = g - jnp.max(g)
    e = jnp.exp(g)
    s = jnp.sum(e)
    o_ref[...] = jnp.sum((e / s) * h, axis=0, keepdims=True)


def _attention_pool(h, w_gate):
    n, d = h.shape
    return pl.pallas_call(
        _pool_kernel,
        grid=(1,),
        in_specs=[
            pl.BlockSpec((n, d), lambda i: (0, 0)),
            pl.BlockSpec((1, d), lambda i: (0, 0)),
        ],
        out_specs=pl.BlockSpec((1, d), lambda i: (0, 0)),
        out_shape=jax.ShapeDtypeStruct((1, d), jnp.float32),
    )(h, w_gate.reshape(1, d))


def kernel(node_feat, edge_feat, eig_vec, edge_index, W_pre, b_pre,
           W_post, b_post, W_gate, b_gate):
    n, d = node_feat.shape
    e, de = edge_feat.shape
    num_layers = W_pre.shape[0]
    delta = 3.5

    src = edge_index[0]
    dst = edge_index[1]

    ones = jnp.ones((e,), jnp.float32)
    B = eig_vec[src, 1] - eig_vec[dst, 1]
    absB = jnp.abs(B)
    wstack = jnp.stack([ones, absB, B], axis=1)
    ef_w = jnp.concatenate(
        [wstack, edge_feat, absB[:, None] * edge_feat, B[:, None] * edge_feat],
        axis=1)
    stats = jax.ops.segment_sum(ef_w, dst, num_segments=n)
    deg = stats[:, 0]
    s_absB = stats[:, 1]
    s_B = stats[:, 2]
    F0 = stats[:, 3:3 + de]
    Fa = stats[:, 3 + de:3 + 2 * de]
    Fb = stats[:, 3 + 2 * de:3 + 3 * de]
    deg_c = jnp.maximum(deg, 1.0)
    sum_absB = s_absB + 1e-30
    amp = jnp.log(deg + 1.0) / delta
    has_edge = deg > 0.0

    h = node_feat
    for l in range(num_layers):
        W1 = W_pre[l, :d]
        W2 = W_pre[l, d:2 * d]
        W3 = W_pre[l, 2 * d:]
        bp = b_pre[l]

        PQ = _matmul(h, jnp.concatenate([W1, W2], axis=1), 2000)
        P = PQ[:, :d]
        Q = PQ[:, d:]
        C = _matmul(edge_feat, W3, 8000)

        V = jnp.take(P, src, axis=0)
        S0 = jax.ops.segment_sum(V, dst, num_segments=n)
        Sa = jax.ops.segment_sum(absB[:, None] * V, dst, num_segments=n)
        Sb = jax.ops.segment_sum(B[:, None] * V, dst, num_segments=n)
        M = jax.ops.segment_max(V + C, dst, num_segments=n)

        FW0 = _matmul(jnp.concatenate([F0, Fa, Fb], axis=0), W3, 2000)
        f0w, faw, fbw = FW0[:n], FW0[n:2 * n], FW0[2 * n:]
        s_sum = S0 + deg[:, None] * Q + f0w + deg[:, None] * bp[None, :]
        agg_mean = s_sum / deg_c[:, None]
        agg_max = jnp.where(has_edge[:, None], M + Q + bp[None, :], 0.0)
        agg_dav = (Sa + s_absB[:, None] * Q + faw
                   + s_absB[:, None] * bp[None, :]) / sum_absB[:, None]
        agg_ddx = (Sb + s_B[:, None] * Q + fbw
                   + s_B[:, None] * bp[None, :]) / sum_absB[:, None]

        aggs = jnp.concatenate([agg_mean, agg_max, agg_dav, agg_ddx], axis=1)
        x = jnp.concatenate([h, aggs, aggs * amp[:, None]], axis=1)
        h = _matmul(x, W_post[l], 2000) + b_post[l][None, :]

    return _attention_pool(h, W_gate[:, 0]) + 0.0 * b_gate[0]


# speedup vs baseline: 1.1870x; 1.0560x over previous
import functools

import jax
import jax.numpy as jnp
from jax.experimental import pallas as pl


def _mm_kernel(x_ref, w_ref, o_ref):
    o_ref[...] = jnp.dot(x_ref[...], w_ref[...],
                         preferred_element_type=jnp.float32)


def _matmul(x, w, block_rows):
    m, k = x.shape
    _, n = w.shape
    grid = m // block_rows
    return pl.pallas_call(
        _mm_kernel,
        grid=(grid,),
        in_specs=[
            pl.BlockSpec((block_rows, k), lambda i: (i, 0)),
            pl.BlockSpec((k, n), lambda i: (0, 0)),
        ],
        out_specs=pl.BlockSpec((block_rows, n), lambda i: (i, 0)),
        out_shape=jax.ShapeDtypeStruct((m, n), jnp.float32),
    )(x, w)


def _pool_kernel(h_ref, wg_ref, o_ref):
    h = h_ref[...]
    g = jnp.sum(h * wg_ref[...], axis=1, keepdims=True)
    g = g - jnp.max(g)
    e = jnp.exp(g)
    s = jnp.sum(e)
    o_ref[...] = jnp.sum((e / s) * h, axis=0, keepdims=True)


def _attention_pool(h, w_gate):
    n, d = h.shape
    return pl.pallas_call(
        _pool_kernel,
        grid=(1,),
        in_specs=[
            pl.BlockSpec((n, d), lambda i: (0, 0)),
            pl.BlockSpec((1, d), lambda i: (0, 0)),
        ],
        out_specs=pl.BlockSpec((1, d), lambda i: (0, 0)),
        out_shape=jax.ShapeDtypeStruct((1, d), jnp.float32),
    )(h, w_gate.reshape(1, d))


def kernel(node_feat, edge_feat, eig_vec, edge_index, W_pre, b_pre,
           W_post, b_post, W_gate, b_gate):
    n, d = node_feat.shape
    e, de = edge_feat.shape
    num_layers = W_pre.shape[0]
    delta = 3.5

    src = edge_index[0]
    dst = edge_index[1]

    ones = jnp.ones((e,), jnp.float32)
    B = eig_vec[src, 1] - eig_vec[dst, 1]
    absB = jnp.abs(B)
    wstack = jnp.stack([ones, absB, B], axis=1)
    ef_w = jnp.concatenate(
        [wstack, edge_feat, absB[:, None] * edge_feat, B[:, None] * edge_feat],
        axis=1)
    stats = jax.ops.segment_sum(ef_w, dst, num_segments=n)
    deg = stats[:, 0]
    s_absB = stats[:, 1]
    s_B = stats[:, 2]
    F0 = stats[:, 3:3 + de]
    Fa = stats[:, 3 + de:3 + 2 * de]
    Fb = stats[:, 3 + 2 * de:3 + 3 * de]
    deg_c = jnp.maximum(deg, 1.0)
    sum_absB = s_absB + 1e-30
    amp = jnp.log(deg + 1.0) / delta
    has_edge = deg > 0.0

    h = node_feat
    for l in range(num_layers):
        W1 = W_pre[l, :d]
        W2 = W_pre[l, d:2 * d]
        W3 = W_pre[l, 2 * d:]
        bp = b_pre[l]

        PQ = _matmul(h, jnp.concatenate([W1, W2], axis=1), 2000)
        P = PQ[:, :d]
        Q = PQ[:, d:]
        C = _matmul(edge_feat, W3, 8000)

        V = jnp.take(P, src, axis=0)
        Vw = jnp.concatenate(
            [V, absB[:, None] * V, B[:, None] * V], axis=1)
        S = jax.ops.segment_sum(Vw, dst, num_segments=n)
        S0, Sa, Sb = S[:, :d], S[:, d:2 * d], S[:, 2 * d:]
        M = jax.ops.segment_max(V + C, dst, num_segments=n)

        FW0 = _matmul(jnp.concatenate([F0, Fa, Fb], axis=0), W3, 2000)
        f0w, faw, fbw = FW0[:n], FW0[n:2 * n], FW0[2 * n:]
        s_sum = S0 + deg[:, None] * Q + f0w + deg[:, None] * bp[None, :]
        agg_mean = s_sum / deg_c[:, None]
        agg_max = jnp.where(has_edge[:, None], M + Q + bp[None, :], 0.0)
        agg_dav = (Sa + s_absB[:, None] * Q + faw
                   + s_absB[:, None] * bp[None, :]) / sum_absB[:, None]
        agg_ddx = (Sb + s_B[:, None] * Q + fbw
                   + s_B[:, None] * bp[None, :]) / sum_absB[:, None]

        aggs = jnp.concatenate([agg_mean, agg_max, agg_dav, agg_ddx], axis=1)
        x = jnp.concatenate([h, aggs, aggs * amp[:, None]], axis=1)
        h = _matmul(x, W_post[l], 2000) + b_post[l][None, :]

    return _attention_pool(h, W_gate[:, 0]) + 0.0 * b_gate[0]


# speedup vs baseline: 1.4485x; 1.2203x over previous
import functools

import jax
import jax.numpy as jnp
from jax.experimental import pallas as pl
from jax.experimental.pallas import tpu as pltpu


def _mm_kernel(x_ref, w_ref, o_ref):
    o_ref[...] = jnp.dot(x_ref[...], w_ref[...],
                         preferred_element_type=jnp.float32)


def _matmul(x, w, block_rows):
    m, k = x.shape
    _, n = w.shape
    grid = m // block_rows
    return pl.pallas_call(
        _mm_kernel,
        grid=(grid,),
        in_specs=[
            pl.BlockSpec((block_rows, k), lambda i: (i, 0)),
            pl.BlockSpec((k, n), lambda i: (0, 0)),
        ],
        out_specs=pl.BlockSpec((block_rows, n), lambda i: (i, 0)),
        out_shape=jax.ShapeDtypeStruct((m, n), jnp.float32),
    )(x, w)


def _mm_acc_kernel(x_ref, w_ref, o_ref):
    @pl.when(pl.program_id(1) == 0)
    def _init():
        o_ref[...] = jnp.zeros_like(o_ref)
    o_ref[...] += jnp.dot(x_ref[...], w_ref[...],
                          preferred_element_type=jnp.float32)


def _matmul_bigk(x, w, block_rows, block_k):
    m, k = x.shape
    _, n = w.shape
    return pl.pallas_call(
        _mm_acc_kernel,
        grid=(m // block_rows, k // block_k),
        in_specs=[
            pl.BlockSpec((block_rows, block_k), lambda i, j: (i, j)),
            pl.BlockSpec((block_k, n), lambda i, j: (j, 0)),
        ],
        out_specs=pl.BlockSpec((block_rows, n), lambda i, j: (i, 0)),
        out_shape=jax.ShapeDtypeStruct((m, n), jnp.float32),
        compiler_params=pltpu.CompilerParams(
            dimension_semantics=("parallel", "arbitrary")),
    )(x, w)


def _pool_kernel(h_ref, wg_ref, o_ref):
    h = h_ref[...]
    g = jnp.sum(h * wg_ref[...], axis=1, keepdims=True)
    g = g - jnp.max(g)
    e = jnp.exp(g)
    s = jnp.sum(e)
    o_ref[...] = jnp.sum((e / s) * h, axis=0, keepdims=True)


def _attention_pool(h, w_gate):
    n, d = h.shape
    return pl.pallas_call(
        _pool_kernel,
        grid=(1,),
        in_specs=[
            pl.BlockSpec((n, d), lambda i: (0, 0)),
            pl.BlockSpec((1, d), lambda i: (0, 0)),
        ],
        out_specs=pl.BlockSpec((1, d), lambda i: (0, 0)),
        out_shape=jax.ShapeDtypeStruct((1, d), jnp.float32),
    )(h, w_gate.reshape(1, d))


def kernel(node_feat, edge_feat, eig_vec, edge_index, W_pre, b_pre,
           W_post, b_post, W_gate, b_gate):
    n, d = node_feat.shape
    e, de = edge_feat.shape
    num_layers = W_pre.shape[0]
    delta = 3.5

    src = edge_index[0]
    dst = edge_index[1]

    ones = jnp.ones((e,), jnp.float32)
    B = eig_vec[src, 1] - eig_vec[dst, 1]
    absB = jnp.abs(B)
    wstack = jnp.stack([ones, absB, B], axis=1)
    ef_w = jnp.concatenate(
        [wstack, edge_feat, absB[:, None] * edge_feat, B[:, None] * edge_feat],
        axis=1)
    stats = jax.ops.segment_sum(ef_w, dst, num_segments=n)
    deg = stats[:, 0]
    s_absB = stats[:, 1]
    s_B = stats[:, 2]
    F0 = stats[:, 3:3 + de]
    Fa = stats[:, 3 + de:3 + 2 * de]
    Fb = stats[:, 3 + 2 * de:3 + 3 * de]
    deg_c = jnp.maximum(deg, 1.0)
    sum_absB = s_absB + 1e-30
    amp = jnp.log(deg + 1.0) / delta
    has_edge = deg > 0.0

    n_pad = ((n + 2559) // 2560) * 2560
    rows = jnp.concatenate([dst, dst + n, dst + 2 * n])
    cols = jnp.concatenate([src, src, src])
    vals = jnp.concatenate([ones, absB, B])
    A = jnp.zeros((3 * n, n_pad), jnp.float32).at[rows, cols].add(vals)

    h = node_feat
    for l in range(num_layers):
        W1 = W_pre[l, :d]
        W2 = W_pre[l, d:2 * d]
        W3 = W_pre[l, 2 * d:]
        bp = b_pre[l]

        PQ = _matmul(h, jnp.concatenate([W1, W2], axis=1), 2000)
        P = PQ[:, :d]
        Q = PQ[:, d:]
        C = _matmul(edge_feat, W3, 8000)

        P_pad = jnp.concatenate(
            [P, jnp.zeros((n_pad - n, d), jnp.float32)], axis=0)
        S = _matmul_bigk(A, P_pad, 1000, 2560)
        S0, Sa, Sb = S[:n], S[n:2 * n], S[2 * n:]
        V = jnp.take(P, src, axis=0)
        M = jax.ops.segment_max(V + C, dst, num_segments=n)

        FW0 = _matmul(jnp.concatenate([F0, Fa, Fb], axis=0), W3, 2000)
        f0w, faw, fbw = FW0[:n], FW0[n:2 * n], FW0[2 * n:]
        s_sum = S0 + deg[:, None] * Q + f0w + deg[:, None] * bp[None, :]
        agg_mean = s_sum / deg_c[:, None]
        agg_max = jnp.where(has_edge[:, None], M + Q + bp[None, :], 0.0)
        agg_dav = (Sa + s_absB[:, None] * Q + faw
                   + s_absB[:, None] * bp[None, :]) / sum_absB[:, None]
        agg_ddx = (Sb + s_B[:, None] * Q + fbw
                   + s_B[:, None] * bp[None, :]) / sum_absB[:, None]

        aggs = jnp.concatenate([agg_mean, agg_max, agg_dav, agg_ddx], axis=1)
        x = jnp.concatenate([h, aggs, aggs * amp[:, None]], axis=1)
        h = _matmul(x, W_post[l], 2000) + b_post[l][None, :]

    return _attention_pool(h, W_gate[:, 0]) + 0.0 * b_gate[0]
